# Initial kernel scaffold; baseline (speedup 1.0000x reference)
import functools

import jax
import jax.numpy as jnp
from jax import lax
from jax.experimental import pallas as pl
from jax.experimental.pallas import tpu as pltpu
from jax.experimental.pallas import tpu_sc as plsc

N = 10000
E = 320000
DIN = 128
DH = 32
STATE_DIM = 64
LSTM_H = 256
ACT_DIM = 64

NC = 2
NS = 16
NW = NC * NS
CHUNK = 128
NCH = 80
E_TILE = NCH * CHUNK
E_PAD = NW * E_TILE
N_PAD = 10240
STRIPE = N_PAD // NS

_MESH = plsc.VectorSubcoreMesh(core_axis_name="c", subcore_axis_name="s")


@functools.partial(
    pl.kernel,
    out_type=jax.ShapeDtypeStruct((NC, N_PAD), jnp.float32),
    mesh=_MESH,
    scratch_types=[
        pltpu.VMEM((NCH, CHUNK), jnp.int32),
        pltpu.VMEM((NCH, CHUNK), jnp.float32),
        pltpu.VMEM_SHARED((N_PAD,), jnp.float32),
    ],
)
def _deg_call(dsts_hbm, ews_hbm, zeros_hbm, out_hbm, dst_v, ew_v, deg_sh):
    cid = lax.axis_index("c")
    sid = lax.axis_index("s")
    wid = cid * NS + sid
    pltpu.sync_copy(zeros_hbm.at[pl.ds(sid * STRIPE, STRIPE)],
                    deg_sh.at[pl.ds(sid * STRIPE, STRIPE)])
    pltpu.sync_copy(dsts_hbm.at[wid], dst_v)
    pltpu.sync_copy(ews_hbm.at[wid], ew_v)
    plsc.subcore_barrier()

    def body(j, _):
        pltpu.sync_copy(ew_v.at[j], deg_sh.at[dst_v.at[j]], add=True)
        return 0

    lax.fori_loop(0, NCH, body, 0)
    plsc.subcore_barrier()
    pltpu.sync_copy(deg_sh.at[pl.ds(sid * STRIPE, STRIPE)],
                    out_hbm.at[cid, pl.ds(sid * STRIPE, STRIPE)])


@functools.partial(
    pl.kernel,
    out_type=jax.ShapeDtypeStruct((NC, N_PAD, DH), jnp.float32),
    mesh=_MESH,
    scratch_types=[
        pltpu.VMEM((NCH, CHUNK), jnp.int32),
        pltpu.VMEM((NCH, CHUNK), jnp.int32),
        pltpu.VMEM((NCH, CHUNK), jnp.float32),
        pltpu.VMEM((CHUNK, DH), jnp.float32),
        pltpu.VMEM((CHUNK, DH), jnp.float32),
        pltpu.VMEM_SHARED((N_PAD, DH), jnp.float32),
        pltpu.SemaphoreType.DMA,
        pltpu.SemaphoreType.DMA,
    ],
)
def _msg_call(g_hbm, srcs_hbm, dsts_hbm, ews_hbm, zeros_hbm, out_hbm,
              src_v, dst_v, ew_v, rows_a, rows_b, acc_sh, sem_a, sem_b):
    cid = lax.axis_index("c")
    sid = lax.axis_index("s")
    wid = cid * NS + sid

    pltpu.sync_copy(zeros_hbm.at[pl.ds(sid * STRIPE, STRIPE)],
                    acc_sh.at[pl.ds(sid * STRIPE, STRIPE)])
    pltpu.sync_copy(srcs_hbm.at[wid], src_v)
    pltpu.sync_copy(dsts_hbm.at[wid], dst_v)
    pltpu.sync_copy(ews_hbm.at[wid], ew_v)
    plsc.subcore_barrier()

    def start_gather(j, rows, sem):
        pltpu.make_async_copy(g_hbm.at[src_v.at[j]], rows, sem).start()

    def wait_gather(j, rows, sem):
        pltpu.make_async_copy(g_hbm.at[src_v.at[j]], rows, sem).wait()

    def scale(rows, j):
        for i in range(CHUNK):
            w = ew_v[j, i]
            wv = jnp.full((16,), w, dtype=jnp.float32)
            rows[i, 0:16] = rows[i, 0:16] * wv
            rows[i, 16:32] = rows[i, 16:32] * wv

    start_gather(0, rows_a, sem_a)
    start_gather(1, rows_b, sem_b)

    def body(t, _):
        ja = 2 * t
        jb = 2 * t + 1
        wait_gather(ja, rows_a, sem_a)
        scale(rows_a, ja)
        pltpu.sync_copy(rows_a, acc_sh.at[dst_v.at[ja]], add=True)

        @pl.when(t < NCH // 2 - 1)
        def _():
            start_gather(ja + 2, rows_a, sem_a)

        wait_gather(jb, rows_b, sem_b)
        scale(rows_b, jb)
        pltpu.sync_copy(rows_b, acc_sh.at[dst_v.at[jb]], add=True)

        @pl.when(t < NCH // 2 - 1)
        def _():
            start_gather(jb + 2, rows_b, sem_b)

        return 0

    lax.fori_loop(0, NCH // 2, body, 0)
    plsc.subcore_barrier()
    pltpu.sync_copy(acc_sh.at[pl.ds(sid * STRIPE, STRIPE)],
                    out_hbm.at[cid, pl.ds(sid * STRIPE, STRIPE)])


BA = 2000


def _dense_a_body(x_ref, w1_ref, degpt_ref, h1_ref, g1_ref, dis_ref, inv_ref):
    deg = jnp.sum(degpt_ref[...], axis=1, keepdims=True) + 1.0
    dis = lax.rsqrt(deg)
    inv = 1.0 / deg
    h1 = jnp.dot(x_ref[...], w1_ref[...], preferred_element_type=jnp.float32)
    h1_ref[...] = h1
    g1_ref[...] = h1 * dis
    dis_ref[...] = dis
    inv_ref[...] = inv


_dense_a = pl.pallas_call(
    _dense_a_body,
    grid=(N // BA,),
    in_specs=[
        pl.BlockSpec((BA, DIN), lambda i: (i, 0)),
        pl.BlockSpec((DIN, DH), lambda i: (0, 0)),
        pl.BlockSpec((BA, NC), lambda i: (i, 0)),
    ],
    out_specs=[
        pl.BlockSpec((BA, DH), lambda i: (i, 0)),
        pl.BlockSpec((BA, DH), lambda i: (i, 0)),
        pl.BlockSpec((BA, 1), lambda i: (i, 0)),
        pl.BlockSpec((BA, 1), lambda i: (i, 0)),
    ],
    out_shape=[
        jax.ShapeDtypeStruct((N, DH), jnp.float32),
        jax.ShapeDtypeStruct((N, DH), jnp.float32),
        jax.ShapeDtypeStruct((N, 1), jnp.float32),
        jax.ShapeDtypeStruct((N, 1), jnp.float32),
    ],
)


def _dense_b_body(accp_ref, h_ref, dis_ref, inv_ref, b_ref, w2_ref,
                  h2_ref, g2_ref):
    acc = accp_ref[0] + accp_ref[1]
    x1 = jnp.maximum(acc * dis_ref[...] + h_ref[...] * inv_ref[...] + b_ref[...], 0.0)
    h2 = jnp.dot(x1, w2_ref[...], preferred_element_type=jnp.float32)
    h2_ref[...] = h2
    g2_ref[...] = h2 * dis_ref[...]


_dense_b = pl.pallas_call(
    _dense_b_body,
    grid=(N // BA,),
    in_specs=[
        pl.BlockSpec((NC, BA, DH), lambda i: (0, i, 0)),
        pl.BlockSpec((BA, DH), lambda i: (i, 0)),
        pl.BlockSpec((BA, 1), lambda i: (i, 0)),
        pl.BlockSpec((BA, 1), lambda i: (i, 0)),
        pl.BlockSpec((1, DH), lambda i: (0, 0)),
        pl.BlockSpec((DH, DH), lambda i: (0, 0)),
    ],
    out_specs=[
        pl.BlockSpec((BA, DH), lambda i: (i, 0)),
        pl.BlockSpec((BA, DH), lambda i: (i, 0)),
    ],
    out_shape=[
        jax.ShapeDtypeStruct((N, DH), jnp.float32),
        jax.ShapeDtypeStruct((N, DH), jnp.float32),
    ],
)


def _dense_c_body(accp_ref, h_ref, dis_ref, inv_ref, b_ref, xs_ref, h0_ref,
                  c0_ref, wih_ref, whh_ref, bih_ref, bhh_ref, wfc_ref,
                  bfc_ref, xo_ref, h1o_ref, c1o_ref, accv):
    i = pl.program_id(0)

    @pl.when(i == 0)
    def _():
        accv[...] = jnp.zeros_like(accv)

    acc = accp_ref[0] + accp_ref[1]
    x2 = jnp.maximum(acc * dis_ref[...] + h_ref[...] * inv_ref[...] + b_ref[...], 0.0)
    accv[...] += jnp.sum(x2, axis=0, keepdims=True)

    @pl.when(i == N // BA - 1)
    def _():
        xg = accv[...] * (1.0 / N)
        xc = jnp.concatenate([xg, xs_ref[...]], axis=1)
        cdims = (((1,), (1,)), ((), ()))
        gates = (lax.dot_general(xc, wih_ref[...], cdims,
                                 preferred_element_type=jnp.float32)
                 + bih_ref[...]
                 + lax.dot_general(h0_ref[...], whh_ref[...], cdims,
                                   preferred_element_type=jnp.float32)
                 + bhh_ref[...])
        H = LSTM_H
        gi = jax.nn.sigmoid(gates[:, 0:H])
        gf = jax.nn.sigmoid(gates[:, H:2 * H])
        gg = jnp.tanh(gates[:, 2 * H:3 * H])
        go = jax.nn.sigmoid(gates[:, 3 * H:4 * H])
        c1 = gf * c0_ref[...] + gi * gg
        h1 = go * jnp.tanh(c1)
        logits = lax.dot_general(h1, wfc_ref[...], cdims,
                                 preferred_element_type=jnp.float32) + bfc_ref[...]
        m = jnp.max(logits, axis=1, keepdims=True)
        lse = jnp.log(jnp.sum(jnp.exp(logits - m), axis=1, keepdims=True))
        xo_ref[...] = (logits - m - lse).reshape(1, 1, ACT_DIM)
        h1o_ref[...] = h1.reshape(1, 1, LSTM_H)
        c1o_ref[...] = c1.reshape(1, 1, LSTM_H)


_dense_c = pl.pallas_call(
    _dense_c_body,
    grid=(N // BA,),
    in_specs=[
        pl.BlockSpec((NC, BA, DH), lambda i: (0, i, 0)),
        pl.BlockSpec((BA, DH), lambda i: (i, 0)),
        pl.BlockSpec((BA, 1), lambda i: (i, 0)),
        pl.BlockSpec((BA, 1), lambda i: (i, 0)),
        pl.BlockSpec((1, DH), lambda i: (0, 0)),
        pl.BlockSpec((1, STATE_DIM), lambda i: (0, 0)),
        pl.BlockSpec((1, LSTM_H), lambda i: (0, 0)),
        pl.BlockSpec((1, LSTM_H), lambda i: (0, 0)),
        pl.BlockSpec((4 * LSTM_H, DH + STATE_DIM), lambda i: (0, 0)),
        pl.BlockSpec((4 * LSTM_H, LSTM_H), lambda i: (0, 0)),
        pl.BlockSpec((1, 4 * LSTM_H), lambda i: (0, 0)),
        pl.BlockSpec((1, 4 * LSTM_H), lambda i: (0, 0)),
        pl.BlockSpec((ACT_DIM, LSTM_H), lambda i: (0, 0)),
        pl.BlockSpec((1, ACT_DIM), lambda i: (0, 0)),
    ],
    out_specs=[
        pl.BlockSpec((1, 1, ACT_DIM), lambda i: (0, 0, 0)),
        pl.BlockSpec((1, 1, LSTM_H), lambda i: (0, 0, 0)),
        pl.BlockSpec((1, 1, LSTM_H), lambda i: (0, 0, 0)),
    ],
    out_shape=[
        jax.ShapeDtypeStruct((1, 1, ACT_DIM), jnp.float32),
        jax.ShapeDtypeStruct((1, 1, LSTM_H), jnp.float32),
        jax.ShapeDtypeStruct((1, 1, LSTM_H), jnp.float32),
    ],
    scratch_shapes=[pltpu.VMEM((1, DH), jnp.float32)],
)


def kernel(x_graph, edge_index, edge_weight, x_state, h0, c0, W1, b1, W2, b2,
           W_ih, W_hh, b_ih, b_hh, W_fc, b_fc):
    src = edge_index[0]
    dst = edge_index[1]
    npad = E_PAD - E
    pad_src = (jnp.arange(npad, dtype=jnp.int32) % N)
    pad_dst = (jnp.arange(npad, dtype=jnp.int32) % (N_PAD - N)) + N
    srcs = jnp.concatenate([src, pad_src]).reshape(NW, NCH, CHUNK)
    dsts = jnp.concatenate([dst, pad_dst]).reshape(NW, NCH, CHUNK)
    ews = jnp.concatenate(
        [edge_weight, jnp.zeros((npad,), jnp.float32)]).reshape(NW, NCH, CHUNK)
    zeros1 = jnp.zeros((N_PAD,), jnp.float32)
    zeros2 = jnp.zeros((N_PAD, DH), jnp.float32)

    degp = _deg_call(dsts, ews, zeros1)
    degpt = degp.T
    h1, g1, dis, inv = _dense_a(x_graph, W1, degpt)
    accp1 = _msg_call(g1, srcs, dsts, ews, zeros2)
    h2, g2 = _dense_b(accp1, h1, dis, inv, b1.reshape(1, DH), W2)
    accp2 = _msg_call(g2, srcs, dsts, ews, zeros2)
    xo, h1o, c1o = _dense_c(accp2, h2, dis, inv, b2.reshape(1, DH), x_state,
                            h0.reshape(1, LSTM_H), c0.reshape(1, LSTM_H),
                            W_ih, W_hh, b_ih.reshape(1, 4 * LSTM_H),
                            b_hh.reshape(1, 4 * LSTM_H), W_fc,
                            b_fc.reshape(1, ACT_DIM))
    return (xo, h1o, c1o)


# speedup vs baseline: 41.3991x; 41.3991x over previous
import functools

import jax
import jax.numpy as jnp
from jax import lax
from jax.experimental import pallas as pl
from jax.experimental.pallas import tpu as pltpu
from jax.experimental.pallas import tpu_sc as plsc

N = 10000
E = 320000
DIN = 128
DH = 32
STATE_DIM = 64
LSTM_H = 256
ACT_DIM = 64

NC = 2
NS = 16
NW = NC * NS
CHUNK = 128
NCH = 80
E_TILE = NCH * CHUNK
E_PAD = NW * E_TILE
N_PAD = 10240
STRIPE = N_PAD // NS

_MESH = plsc.VectorSubcoreMesh(core_axis_name="c", subcore_axis_name="s")
_SC_PARAMS = pltpu.CompilerParams(use_tc_tiling_on_sc=False)


@functools.partial(
    pl.kernel,
    out_type=jax.ShapeDtypeStruct((NC, N_PAD), jnp.float32),
    mesh=_MESH,
    scratch_types=[
        pltpu.VMEM((NCH, CHUNK), jnp.int32),
        pltpu.VMEM((NCH, CHUNK), jnp.float32),
        pltpu.VMEM_SHARED((N_PAD,), jnp.float32),
    ],
    compiler_params=_SC_PARAMS,
)
def _deg_call(dsts_hbm, ews_hbm, zeros_hbm, out_hbm, dst_v, ew_v, deg_sh):
    cid = lax.axis_index("c")
    sid = lax.axis_index("s")
    wid = cid * NS + sid
    pltpu.sync_copy(zeros_hbm.at[pl.ds(sid * STRIPE, STRIPE)],
                    deg_sh.at[pl.ds(sid * STRIPE, STRIPE)])
    pltpu.sync_copy(dsts_hbm.at[wid], dst_v)
    pltpu.sync_copy(ews_hbm.at[wid], ew_v)
    plsc.subcore_barrier()

    def body(j, _):
        pltpu.sync_copy(ew_v.at[j], deg_sh.at[dst_v.at[j]], add=True)
        return 0

    lax.fori_loop(0, NCH, body, 0)
    plsc.subcore_barrier()
    pltpu.sync_copy(deg_sh.at[pl.ds(sid * STRIPE, STRIPE)],
                    out_hbm.at[cid, pl.ds(sid * STRIPE, STRIPE)])


@functools.partial(
    pl.kernel,
    out_type=jax.ShapeDtypeStruct((NC, N_PAD, DH), jnp.float32),
    mesh=_MESH,
    scratch_types=[
        pltpu.VMEM((NCH, CHUNK), jnp.int32),
        pltpu.VMEM((NCH, CHUNK), jnp.int32),
        pltpu.VMEM((NCH, CHUNK), jnp.float32),
        pltpu.VMEM((CHUNK, DH), jnp.float32),
        pltpu.VMEM((CHUNK, DH), jnp.float32),
        pltpu.VMEM_SHARED((N_PAD, DH), jnp.float32),
        pltpu.SemaphoreType.DMA,
        pltpu.SemaphoreType.DMA,
    ],
    compiler_params=_SC_PARAMS,
)
def _msg_call(g_hbm, srcs_hbm, dsts_hbm, ews_hbm, zeros_hbm, out_hbm,
              src_v, dst_v, ew_v, rows_a, rows_b, acc_sh, sem_a, sem_b):
    cid = lax.axis_index("c")
    sid = lax.axis_index("s")
    wid = cid * NS + sid

    pltpu.sync_copy(zeros_hbm.at[pl.ds(sid * STRIPE, STRIPE)],
                    acc_sh.at[pl.ds(sid * STRIPE, STRIPE)])
    pltpu.sync_copy(srcs_hbm.at[wid], src_v)
    pltpu.sync_copy(dsts_hbm.at[wid], dst_v)
    pltpu.sync_copy(ews_hbm.at[wid], ew_v)
    plsc.subcore_barrier()

    def start_gather(j, rows, sem):
        pltpu.make_async_copy(g_hbm.at[src_v.at[j]], rows, sem).start()

    def wait_gather(j, rows, sem):
        pltpu.make_async_copy(g_hbm.at[src_v.at[j]], rows, sem).wait()

    def scale(rows, j):
        for k in range(CHUNK // 16):
            wv16 = ew_v[j, pl.ds(16 * k, 16)]
            for l in range(16):
                i = 16 * k + l
                wv = jnp.full((16,), wv16[l], dtype=jnp.float32)
                rows[i, 0:16] = rows[i, 0:16] * wv
                rows[i, 16:32] = rows[i, 16:32] * wv

    start_gather(0, rows_a, sem_a)
    start_gather(1, rows_b, sem_b)

    def body(t, _):
        ja = 2 * t
        jb = 2 * t + 1
        wait_gather(ja, rows_a, sem_a)
        scale(rows_a, ja)
        pltpu.sync_copy(rows_a, acc_sh.at[dst_v.at[ja]], add=True)

        @pl.when(t < NCH // 2 - 1)
        def _():
            start_gather(ja + 2, rows_a, sem_a)

        wait_gather(jb, rows_b, sem_b)
        scale(rows_b, jb)
        pltpu.sync_copy(rows_b, acc_sh.at[dst_v.at[jb]], add=True)

        @pl.when(t < NCH // 2 - 1)
        def _():
            start_gather(jb + 2, rows_b, sem_b)

        return 0

    lax.fori_loop(0, NCH // 2, body, 0)
    plsc.subcore_barrier()
    pltpu.sync_copy(acc_sh.at[pl.ds(sid * STRIPE, STRIPE)],
                    out_hbm.at[cid, pl.ds(sid * STRIPE, STRIPE)])


BA = 2000


def _dense_a_body(x_ref, w1_ref, degpt_ref, h1_ref, g1_ref, dis_ref, inv_ref):
    deg = jnp.sum(degpt_ref[...], axis=1, keepdims=True) + 1.0
    dis = lax.rsqrt(deg)
    inv = 1.0 / deg
    h1 = jnp.dot(x_ref[...], w1_ref[...], preferred_element_type=jnp.float32)
    h1_ref[...] = h1
    g1_ref[...] = h1 * dis
    dis_ref[...] = dis
    inv_ref[...] = inv


_dense_a = pl.pallas_call(
    _dense_a_body,
    grid=(N // BA,),
    in_specs=[
        pl.BlockSpec((BA, DIN), lambda i: (i, 0)),
        pl.BlockSpec((DIN, DH), lambda i: (0, 0)),
        pl.BlockSpec((BA, NC), lambda i: (i, 0)),
    ],
    out_specs=[
        pl.BlockSpec((BA, DH), lambda i: (i, 0)),
        pl.BlockSpec((BA, DH), lambda i: (i, 0)),
        pl.BlockSpec((BA, 1), lambda i: (i, 0)),
        pl.BlockSpec((BA, 1), lambda i: (i, 0)),
    ],
    out_shape=[
        jax.ShapeDtypeStruct((N, DH), jnp.float32),
        jax.ShapeDtypeStruct((N, DH), jnp.float32),
        jax.ShapeDtypeStruct((N, 1), jnp.float32),
        jax.ShapeDtypeStruct((N, 1), jnp.float32),
    ],
)


def _dense_b_body(accp_ref, h_ref, dis_ref, inv_ref, b_ref, w2_ref,
                  h2_ref, g2_ref):
    acc = accp_ref[0] + accp_ref[1]
    x1 = jnp.maximum(acc * dis_ref[...] + h_ref[...] * inv_ref[...] + b_ref[...], 0.0)
    h2 = jnp.dot(x1, w2_ref[...], preferred_element_type=jnp.float32)
    h2_ref[...] = h2
    g2_ref[...] = h2 * dis_ref[...]


_dense_b = pl.pallas_call(
    _dense_b_body,
    grid=(N // BA,),
    in_specs=[
        pl.BlockSpec((NC, BA, DH), lambda i: (0, i, 0)),
        pl.BlockSpec((BA, DH), lambda i: (i, 0)),
        pl.BlockSpec((BA, 1), lambda i: (i, 0)),
        pl.BlockSpec((BA, 1), lambda i: (i, 0)),
        pl.BlockSpec((1, DH), lambda i: (0, 0)),
        pl.BlockSpec((DH, DH), lambda i: (0, 0)),
    ],
    out_specs=[
        pl.BlockSpec((BA, DH), lambda i: (i, 0)),
        pl.BlockSpec((BA, DH), lambda i: (i, 0)),
    ],
    out_shape=[
        jax.ShapeDtypeStruct((N, DH), jnp.float32),
        jax.ShapeDtypeStruct((N, DH), jnp.float32),
    ],
)


def _dense_c_body(accp_ref, h_ref, dis_ref, inv_ref, b_ref, xs_ref, h0_ref,
                  c0_ref, wih_ref, whh_ref, bih_ref, bhh_ref, wfc_ref,
                  bfc_ref, xo_ref, h1o_ref, c1o_ref, accv):
    i = pl.program_id(0)

    @pl.when(i == 0)
    def _():
        accv[...] = jnp.zeros_like(accv)

    acc = accp_ref[0] + accp_ref[1]
    x2 = jnp.maximum(acc * dis_ref[...] + h_ref[...] * inv_ref[...] + b_ref[...], 0.0)
    accv[...] += jnp.sum(x2, axis=0, keepdims=True)

    @pl.when(i == N // BA - 1)
    def _():
        xg = accv[...] * (1.0 / N)
        xc = jnp.concatenate([xg, xs_ref[...]], axis=1)
        cdims = (((1,), (1,)), ((), ()))
        gates = (lax.dot_general(xc, wih_ref[...], cdims,
                                 preferred_element_type=jnp.float32)
                 + bih_ref[...]
                 + lax.dot_general(h0_ref[...], whh_ref[...], cdims,
                                   preferred_element_type=jnp.float32)
                 + bhh_ref[...])
        H = LSTM_H
        gi = jax.nn.sigmoid(gates[:, 0:H])
        gf = jax.nn.sigmoid(gates[:, H:2 * H])
        gg = jnp.tanh(gates[:, 2 * H:3 * H])
        go = jax.nn.sigmoid(gates[:, 3 * H:4 * H])
        c1 = gf * c0_ref[...] + gi * gg
        h1 = go * jnp.tanh(c1)
        logits = lax.dot_general(h1, wfc_ref[...], cdims,
                                 preferred_element_type=jnp.float32) + bfc_ref[...]
        m = jnp.max(logits, axis=1, keepdims=True)
        lse = jnp.log(jnp.sum(jnp.exp(logits - m), axis=1, keepdims=True))
        xo_ref[...] = (logits - m - lse).reshape(1, 1, ACT_DIM)
        h1o_ref[...] = h1.reshape(1, 1, LSTM_H)
        c1o_ref[...] = c1.reshape(1, 1, LSTM_H)


_dense_c = pl.pallas_call(
    _dense_c_body,
    grid=(N // BA,),
    in_specs=[
        pl.BlockSpec((NC, BA, DH), lambda i: (0, i, 0)),
        pl.BlockSpec((BA, DH), lambda i: (i, 0)),
        pl.BlockSpec((BA, 1), lambda i: (i, 0)),
        pl.BlockSpec((BA, 1), lambda i: (i, 0)),
        pl.BlockSpec((1, DH), lambda i: (0, 0)),
        pl.BlockSpec((1, STATE_DIM), lambda i: (0, 0)),
        pl.BlockSpec((1, LSTM_H), lambda i: (0, 0)),
        pl.BlockSpec((1, LSTM_H), lambda i: (0, 0)),
        pl.BlockSpec((4 * LSTM_H, DH + STATE_DIM), lambda i: (0, 0)),
        pl.BlockSpec((4 * LSTM_H, LSTM_H), lambda i: (0, 0)),
        pl.BlockSpec((1, 4 * LSTM_H), lambda i: (0, 0)),
        pl.BlockSpec((1, 4 * LSTM_H), lambda i: (0, 0)),
        pl.BlockSpec((ACT_DIM, LSTM_H), lambda i: (0, 0)),
        pl.BlockSpec((1, ACT_DIM), lambda i: (0, 0)),
    ],
    out_specs=[
        pl.BlockSpec((1, 1, ACT_DIM), lambda i: (0, 0, 0)),
        pl.BlockSpec((1, 1, LSTM_H), lambda i: (0, 0, 0)),
        pl.BlockSpec((1, 1, LSTM_H), lambda i: (0, 0, 0)),
    ],
    out_shape=[
        jax.ShapeDtypeStruct((1, 1, ACT_DIM), jnp.float32),
        jax.ShapeDtypeStruct((1, 1, LSTM_H), jnp.float32),
        jax.ShapeDtypeStruct((1, 1, LSTM_H), jnp.float32),
    ],
    scratch_shapes=[pltpu.VMEM((1, DH), jnp.float32)],
)


def kernel(x_graph, edge_index, edge_weight, x_state, h0, c0, W1, b1, W2, b2,
           W_ih, W_hh, b_ih, b_hh, W_fc, b_fc):
    src = edge_index[0]
    dst = edge_index[1]
    npad = E_PAD - E
    pad_src = (jnp.arange(npad, dtype=jnp.int32) % N)
    pad_dst = (jnp.arange(npad, dtype=jnp.int32) % (N_PAD - N)) + N
    srcs = jnp.concatenate([src, pad_src]).reshape(NW, NCH, CHUNK)
    dsts = jnp.concatenate([dst, pad_dst]).reshape(NW, NCH, CHUNK)
    ews = jnp.concatenate(
        [edge_weight, jnp.zeros((npad,), jnp.float32)]).reshape(NW, NCH, CHUNK)
    zeros1 = jnp.zeros((N_PAD,), jnp.float32)
    zeros2 = jnp.zeros((N_PAD, DH), jnp.float32)

    degp = _deg_call(dsts, ews, zeros1)
    degpt = degp.T
    h1, g1, dis, inv = _dense_a(x_graph, W1, degpt)
    accp1 = _msg_call(g1, srcs, dsts, ews, zeros2)
    h2, g2 = _dense_b(accp1, h1, dis, inv, b1.reshape(1, DH), W2)
    accp2 = _msg_call(g2, srcs, dsts, ews, zeros2)
    xo, h1o, c1o = _dense_c(accp2, h2, dis, inv, b2.reshape(1, DH), x_state,
                            h0.reshape(1, LSTM_H), c0.reshape(1, LSTM_H),
                            W_ih, W_hh, b_ih.reshape(1, 4 * LSTM_H),
                            b_hh.reshape(1, 4 * LSTM_H), W_fc,
                            b_fc.reshape(1, ACT_DIM))
    return (xo, h1o, c1o)


# speedup vs baseline: 43.8050x; 1.0581x over previous
import functools

import jax
import jax.numpy as jnp
from jax import lax
from jax.experimental import pallas as pl
from jax.experimental.pallas import tpu as pltpu
from jax.experimental.pallas import tpu_sc as plsc

N = 10000
E = 320000
DIN = 128
DH = 32
STATE_DIM = 64
LSTM_H = 256
ACT_DIM = 64

NC = 2
NS = 16
NW = NC * NS
CHUNK = 128
NCH = 80
E_TILE = NCH * CHUNK
E_PAD = NW * E_TILE
N_PAD = 10240
STRIPE = N_PAD // NS

_MESH = plsc.VectorSubcoreMesh(core_axis_name="c", subcore_axis_name="s")
_SC_PARAMS = pltpu.CompilerParams(use_tc_tiling_on_sc=False)


@functools.partial(
    pl.kernel,
    out_type=jax.ShapeDtypeStruct((NC, N_PAD), jnp.float32),
    mesh=_MESH,
    scratch_types=[
        pltpu.VMEM((NCH, CHUNK), jnp.int32),
        pltpu.VMEM((NCH, CHUNK), jnp.float32),
        pltpu.VMEM_SHARED((N_PAD,), jnp.float32),
    ],
    compiler_params=_SC_PARAMS,
)
def _deg_call(dsts_hbm, ews_hbm, zeros_hbm, out_hbm, dst_v, ew_v, deg_sh):
    cid = lax.axis_index("c")
    sid = lax.axis_index("s")
    wid = cid * NS + sid
    pltpu.sync_copy(zeros_hbm.at[pl.ds(sid * STRIPE, STRIPE)],
                    deg_sh.at[pl.ds(sid * STRIPE, STRIPE)])
    pltpu.sync_copy(dsts_hbm.at[wid], dst_v)
    pltpu.sync_copy(ews_hbm.at[wid], ew_v)
    plsc.subcore_barrier()

    def body(j, _):
        pltpu.sync_copy(ew_v.at[j], deg_sh.at[dst_v.at[j]], add=True)
        return 0

    lax.fori_loop(0, NCH, body, 0)
    plsc.subcore_barrier()
    pltpu.sync_copy(deg_sh.at[pl.ds(sid * STRIPE, STRIPE)],
                    out_hbm.at[cid, pl.ds(sid * STRIPE, STRIPE)])


@functools.partial(
    pl.kernel,
    out_type=jax.ShapeDtypeStruct((NC, N_PAD, DH), jnp.float32),
    mesh=_MESH,
    scratch_types=[
        pltpu.VMEM((NCH, CHUNK), jnp.int32),
        pltpu.VMEM((NCH, CHUNK), jnp.int32),
        pltpu.VMEM((NCH, CHUNK), jnp.float32),
        [pltpu.VMEM((CHUNK, DH), jnp.float32) for _ in range(4)],
        pltpu.VMEM_SHARED((N_PAD, DH), jnp.float32),
        [pltpu.SemaphoreType.DMA for _ in range(4)],
        [pltpu.SemaphoreType.DMA for _ in range(4)],
    ],
    compiler_params=_SC_PARAMS,
)
def _msg_call(g_hbm, srcs_hbm, dsts_hbm, ews_hbm, zeros_hbm, out_hbm,
              src_v, dst_v, ew_v, rows, acc_sh, gsem, ssem):
    cid = lax.axis_index("c")
    sid = lax.axis_index("s")
    wid = cid * NS + sid

    pltpu.sync_copy(zeros_hbm.at[pl.ds(sid * STRIPE, STRIPE)],
                    acc_sh.at[pl.ds(sid * STRIPE, STRIPE)])
    pltpu.sync_copy(srcs_hbm.at[wid], src_v)
    pltpu.sync_copy(dsts_hbm.at[wid], dst_v)
    pltpu.sync_copy(ews_hbm.at[wid], ew_v)
    plsc.subcore_barrier()

    def start_gather(j, b):
        pltpu.make_async_copy(g_hbm.at[src_v.at[j]], rows[b], gsem[b]).start()

    def wait_gather(j, b):
        pltpu.make_async_copy(g_hbm.at[src_v.at[j]], rows[b], gsem[b]).wait()

    def start_scatter(j, b):
        pltpu.async_copy(rows[b], acc_sh.at[dst_v.at[j]], ssem[b], add=True)

    def wait_scatter(j, b):
        pltpu.make_async_copy(rows[b], acc_sh.at[dst_v.at[j]], ssem[b]).wait()

    def scale(b, j):
        buf = rows[b]
        for k in range(CHUNK // 16):
            wv16 = ew_v[j, pl.ds(16 * k, 16)]
            for l in range(16):
                i = 16 * k + l
                wv = jnp.full((16,), wv16[l], dtype=jnp.float32)
                buf[i, 0:16] = buf[i, 0:16] * wv
                buf[i, 16:32] = buf[i, 16:32] * wv

    start_gather(0, 0)
    start_gather(1, 1)

    def body(t, _):
        for b in range(4):
            j = 4 * t + b
            wait_gather(j, b)
            scale(b, j)
            start_scatter(j, b)

            @pl.when(j >= 2)
            def _():
                wait_scatter(j - 2, (b - 2) % 4)

            @pl.when(j + 2 < NCH)
            def _():
                start_gather(j + 2, (b + 2) % 4)

        return 0

    lax.fori_loop(0, NCH // 4, body, 0)
    wait_scatter(NCH - 2, (NCH - 2) % 4)
    wait_scatter(NCH - 1, (NCH - 1) % 4)
    plsc.subcore_barrier()
    pltpu.sync_copy(acc_sh.at[pl.ds(sid * STRIPE, STRIPE)],
                    out_hbm.at[cid, pl.ds(sid * STRIPE, STRIPE)])


BA = 2000


def _dense_a_body(x_ref, w1_ref, degpt_ref, h1_ref, g1_ref, dis_ref, inv_ref):
    deg = jnp.sum(degpt_ref[...], axis=1, keepdims=True) + 1.0
    dis = lax.rsqrt(deg)
    inv = 1.0 / deg
    h1 = jnp.dot(x_ref[...], w1_ref[...], preferred_element_type=jnp.float32)
    h1_ref[...] = h1
    g1_ref[...] = h1 * dis
    dis_ref[...] = dis
    inv_ref[...] = inv


_dense_a = pl.pallas_call(
    _dense_a_body,
    grid=(N // BA,),
    in_specs=[
        pl.BlockSpec((BA, DIN), lambda i: (i, 0)),
        pl.BlockSpec((DIN, DH), lambda i: (0, 0)),
        pl.BlockSpec((BA, NC), lambda i: (i, 0)),
    ],
    out_specs=[
        pl.BlockSpec((BA, DH), lambda i: (i, 0)),
        pl.BlockSpec((BA, DH), lambda i: (i, 0)),
        pl.BlockSpec((BA, 1), lambda i: (i, 0)),
        pl.BlockSpec((BA, 1), lambda i: (i, 0)),
    ],
    out_shape=[
        jax.ShapeDtypeStruct((N, DH), jnp.float32),
        jax.ShapeDtypeStruct((N, DH), jnp.float32),
        jax.ShapeDtypeStruct((N, 1), jnp.float32),
        jax.ShapeDtypeStruct((N, 1), jnp.float32),
    ],
)


def _dense_b_body(accp_ref, h_ref, dis_ref, inv_ref, b_ref, w2_ref,
                  h2_ref, g2_ref):
    acc = accp_ref[0] + accp_ref[1]
    x1 = jnp.maximum(acc * dis_ref[...] + h_ref[...] * inv_ref[...] + b_ref[...], 0.0)
    h2 = jnp.dot(x1, w2_ref[...], preferred_element_type=jnp.float32)
    h2_ref[...] = h2
    g2_ref[...] = h2 * dis_ref[...]


_dense_b = pl.pallas_call(
    _dense_b_body,
    grid=(N // BA,),
    in_specs=[
        pl.BlockSpec((NC, BA, DH), lambda i: (0, i, 0)),
        pl.BlockSpec((BA, DH), lambda i: (i, 0)),
        pl.BlockSpec((BA, 1), lambda i: (i, 0)),
        pl.BlockSpec((BA, 1), lambda i: (i, 0)),
        pl.BlockSpec((1, DH), lambda i: (0, 0)),
        pl.BlockSpec((DH, DH), lambda i: (0, 0)),
    ],
    out_specs=[
        pl.BlockSpec((BA, DH), lambda i: (i, 0)),
        pl.BlockSpec((BA, DH), lambda i: (i, 0)),
    ],
    out_shape=[
        jax.ShapeDtypeStruct((N, DH), jnp.float32),
        jax.ShapeDtypeStruct((N, DH), jnp.float32),
    ],
)


def _dense_c_body(accp_ref, h_ref, dis_ref, inv_ref, b_ref, xs_ref, h0_ref,
                  c0_ref, wih_ref, whh_ref, bih_ref, bhh_ref, wfc_ref,
                  bfc_ref, xo_ref, h1o_ref, c1o_ref, accv):
    i = pl.program_id(0)

    @pl.when(i == 0)
    def _():
        accv[...] = jnp.zeros_like(accv)

    acc = accp_ref[0] + accp_ref[1]
    x2 = jnp.maximum(acc * dis_ref[...] + h_ref[...] * inv_ref[...] + b_ref[...], 0.0)
    accv[...] += jnp.sum(x2, axis=0, keepdims=True)

    @pl.when(i == N // BA - 1)
    def _():
        xg = accv[...] * (1.0 / N)
        xc = jnp.concatenate([xg, xs_ref[...]], axis=1)
        cdims = (((1,), (1,)), ((), ()))
        gates = (lax.dot_general(xc, wih_ref[...], cdims,
                                 preferred_element_type=jnp.float32)
                 + bih_ref[...]
                 + lax.dot_general(h0_ref[...], whh_ref[...], cdims,
                                   preferred_element_type=jnp.float32)
                 + bhh_ref[...])
        H = LSTM_H
        gi = jax.nn.sigmoid(gates[:, 0:H])
        gf = jax.nn.sigmoid(gates[:, H:2 * H])
        gg = jnp.tanh(gates[:, 2 * H:3 * H])
        go = jax.nn.sigmoid(gates[:, 3 * H:4 * H])
        c1 = gf * c0_ref[...] + gi * gg
        h1 = go * jnp.tanh(c1)
        logits = lax.dot_general(h1, wfc_ref[...], cdims,
                                 preferred_element_type=jnp.float32) + bfc_ref[...]
        m = jnp.max(logits, axis=1, keepdims=True)
        lse = jnp.log(jnp.sum(jnp.exp(logits - m), axis=1, keepdims=True))
        xo_ref[...] = (logits - m - lse).reshape(1, 1, ACT_DIM)
        h1o_ref[...] = h1.reshape(1, 1, LSTM_H)
        c1o_ref[...] = c1.reshape(1, 1, LSTM_H)


_dense_c = pl.pallas_call(
    _dense_c_body,
    grid=(N // BA,),
    in_specs=[
        pl.BlockSpec((NC, BA, DH), lambda i: (0, i, 0)),
        pl.BlockSpec((BA, DH), lambda i: (i, 0)),
        pl.BlockSpec((BA, 1), lambda i: (i, 0)),
        pl.BlockSpec((BA, 1), lambda i: (i, 0)),
        pl.BlockSpec((1, DH), lambda i: (0, 0)),
        pl.BlockSpec((1, STATE_DIM), lambda i: (0, 0)),
        pl.BlockSpec((1, LSTM_H), lambda i: (0, 0)),
        pl.BlockSpec((1, LSTM_H), lambda i: (0, 0)),
        pl.BlockSpec((4 * LSTM_H, DH + STATE_DIM), lambda i: (0, 0)),
        pl.BlockSpec((4 * LSTM_H, LSTM_H), lambda i: (0, 0)),
        pl.BlockSpec((1, 4 * LSTM_H), lambda i: (0, 0)),
        pl.BlockSpec((1, 4 * LSTM_H), lambda i: (0, 0)),
        pl.BlockSpec((ACT_DIM, LSTM_H), lambda i: (0, 0)),
        pl.BlockSpec((1, ACT_DIM), lambda i: (0, 0)),
    ],
    out_specs=[
        pl.BlockSpec((1, 1, ACT_DIM), lambda i: (0, 0, 0)),
        pl.BlockSpec((1, 1, LSTM_H), lambda i: (0, 0, 0)),
        pl.BlockSpec((1, 1, LSTM_H), lambda i: (0, 0, 0)),
    ],
    out_shape=[
        jax.ShapeDtypeStruct((1, 1, ACT_DIM), jnp.float32),
        jax.ShapeDtypeStruct((1, 1, LSTM_H), jnp.float32),
        jax.ShapeDtypeStruct((1, 1, LSTM_H), jnp.float32),
    ],
    scratch_shapes=[pltpu.VMEM((1, DH), jnp.float32)],
)


def kernel(x_graph, edge_index, edge_weight, x_state, h0, c0, W1, b1, W2, b2,
           W_ih, W_hh, b_ih, b_hh, W_fc, b_fc):
    src = edge_index[0]
    dst = edge_index[1]
    npad = E_PAD - E
    pad_src = (jnp.arange(npad, dtype=jnp.int32) % N)
    pad_dst = (jnp.arange(npad, dtype=jnp.int32) % (N_PAD - N)) + N
    srcs = jnp.concatenate([src, pad_src]).reshape(NW, NCH, CHUNK)
    dsts = jnp.concatenate([dst, pad_dst]).reshape(NW, NCH, CHUNK)
    ews = jnp.concatenate(
        [edge_weight, jnp.zeros((npad,), jnp.float32)]).reshape(NW, NCH, CHUNK)
    zeros1 = jnp.zeros((N_PAD,), jnp.float32)
    zeros2 = jnp.zeros((N_PAD, DH), jnp.float32)

    degp = _deg_call(dsts, ews, zeros1)
    degpt = degp.T
    h1, g1, dis, inv = _dense_a(x_graph, W1, degpt)
    accp1 = _msg_call(g1, srcs, dsts, ews, zeros2)
    h2, g2 = _dense_b(accp1, h1, dis, inv, b1.reshape(1, DH), W2)
    accp2 = _msg_call(g2, srcs, dsts, ews, zeros2)
    xo, h1o, c1o = _dense_c(accp2, h2, dis, inv, b2.reshape(1, DH), x_state,
                            h0.reshape(1, LSTM_H), c0.reshape(1, LSTM_H),
                            W_ih, W_hh, b_ih.reshape(1, 4 * LSTM_H),
                            b_hh.reshape(1, 4 * LSTM_H), W_fc,
                            b_fc.reshape(1, ACT_DIM))
    return (xo, h1o, c1o)
